# baseline (speedup 1.0000x reference)
import functools

import jax
import jax.numpy as jnp
from jax import lax
from jax.experimental import pallas as pl
from jax.experimental.pallas import tpu as pltpu
from jax.experimental.pallas import tpu_sc as plsc

MAX_LEN = 8192
DIM = 1024

_info = plsc.get_sparse_core_info()
_NC, _NS, _L = _info.num_cores, _info.num_subcores, _info.num_lanes
_NW = _NC * _NS
_ROWS_PER_W = MAX_LEN // _NW
_CHUNK = 16
_NCHUNK = _ROWS_PER_W // _CHUNK
_NBUF = 6


def _pe_kernel(clamp_hbm, table_hbm, out_hbm, clamp_v, idx, buf, gsem, wsem):
    wid = lax.axis_index("s") * _NC + lax.axis_index("c")
    base = wid * _ROWS_PER_W

    _NPRE = min(_NBUF - 1, _NCHUNK)
    hpre = [pltpu.async_copy(table_hbm.at[pl.ds(base + b * _CHUNK, _CHUNK)],
                             buf[b], gsem[b]) for b in range(_NPRE)]

    pltpu.sync_copy(clamp_hbm, clamp_v)
    clamp_vec = clamp_v[...]
    iota = lax.iota(jnp.int32, _L)
    clamp_s = clamp_vec[0]

    def run_slab(start_gather, prefetched):
        def start_write(p, c):
            row0 = base + c * _CHUNK
            return pltpu.async_copy(buf[p], out_hbm.at[pl.ds(row0, _CHUNK)],
                                    wsem[p])

        gh = [None] * _NBUF
        wh = [None] * _NBUF
        for b in range(_NPRE):
            gh[b] = hpre[b] if prefetched else start_gather(b, b)
        for c in range(_NCHUNK):
            nxt = c + _NBUF - 1
            if nxt < _NCHUNK:
                q = nxt % _NBUF
                if wh[q] is not None:
                    wh[q].wait()
                    wh[q] = None
                gh[q] = start_gather(q, nxt)
            p = c % _NBUF
            gh[p].wait()
            wh[p] = start_write(p, c)
        for p in range(_NBUF):
            if wh[p] is not None:
                wh[p].wait()

    @pl.when(base + _ROWS_PER_W - 1 <= clamp_s)
    def _identity_slab():
        def start_linear(p, c):
            row0 = base + c * _CHUNK
            return pltpu.async_copy(table_hbm.at[pl.ds(row0, _CHUNK)],
                                    buf[p], gsem[p])

        run_slab(start_linear, prefetched=True)

    @pl.when(base + _ROWS_PER_W - 1 > clamp_s)
    def _clamped_slab():
        for h in hpre:
            h.wait()

        def start_indirect(p, c):
            row0 = base + c * _CHUNK
            for j in range(_CHUNK // _L):
                v = jnp.minimum(iota + (row0 + j * _L), clamp_vec)
                idx[p][pl.ds(j * _L, _L)] = jnp.maximum(v, 0)
            return pltpu.async_copy(table_hbm.at[idx[p]], buf[p], gsem[p])

        run_slab(start_indirect, prefetched=False)


@functools.partial(
    pl.kernel,
    out_type=jax.ShapeDtypeStruct((MAX_LEN, DIM), jnp.float32),
    mesh=plsc.VectorSubcoreMesh(core_axis_name="c", subcore_axis_name="s"),
    scratch_types=(
        [pltpu.VMEM((_L,), jnp.int32)]
        + [pltpu.VMEM((_CHUNK,), jnp.int32) for _ in range(_NBUF)]
        + [pltpu.VMEM((_CHUNK, DIM), jnp.float32) for _ in range(_NBUF)]
        + [pltpu.SemaphoreType.DMA for _ in range(2 * _NBUF)]
    ),
)
def _pe_call(clamp_hbm, table_hbm, out_hbm, clamp_v, *scratch):
    idx = scratch[:_NBUF]
    buf = scratch[_NBUF:2 * _NBUF]
    gsem = scratch[2 * _NBUF:3 * _NBUF]
    wsem = scratch[3 * _NBUF:4 * _NBUF]
    _pe_kernel(clamp_hbm, table_hbm, out_hbm, clamp_v, idx, buf, gsem, wsem)


def kernel(seq_len, table):
    clamp = jnp.full((_L,), jnp.asarray(seq_len, jnp.int32) - 1, jnp.int32)
    return _pe_call(clamp, table)


# speedup vs baseline: 1.0372x; 1.0372x over previous
import functools

import jax
import jax.numpy as jnp
from jax import lax
from jax.experimental import pallas as pl
from jax.experimental.pallas import tpu as pltpu
from jax.experimental.pallas import tpu_sc as plsc

MAX_LEN = 8192
DIM = 1024

_info = plsc.get_sparse_core_info()
_NC, _NS, _L = _info.num_cores, _info.num_subcores, _info.num_lanes
_NW = _NC * _NS
_ROWS_PER_W = MAX_LEN // _NW
_CHUNK = 32
_NCHUNK = _ROWS_PER_W // _CHUNK
_NBUF = 3


def _pe_kernel(clamp_hbm, table_hbm, out_hbm, clamp_v, idx, buf, gsem, wsem):
    wid = lax.axis_index("c") * _NS + lax.axis_index("s")
    base = wid * _ROWS_PER_W

    _NPRE = min(_NBUF - 1, _NCHUNK)
    hpre = [pltpu.async_copy(table_hbm.at[pl.ds(base + b * _CHUNK, _CHUNK)],
                             buf[b], gsem[b]) for b in range(_NPRE)]

    pltpu.sync_copy(clamp_hbm, clamp_v)
    clamp_vec = clamp_v[...]
    iota = lax.iota(jnp.int32, _L)
    clamp_s = clamp_vec[0]

    def run_slab(start_gather, prefetched):
        def start_write(p, c):
            row0 = base + c * _CHUNK
            return pltpu.async_copy(buf[p], out_hbm.at[pl.ds(row0, _CHUNK)],
                                    wsem[p])

        gh = [None] * _NBUF
        wh = [None] * _NBUF
        for b in range(_NPRE):
            gh[b] = hpre[b] if prefetched else start_gather(b, b)
        for c in range(_NCHUNK):
            nxt = c + _NBUF - 1
            if nxt < _NCHUNK:
                q = nxt % _NBUF
                if wh[q] is not None:
                    wh[q].wait()
                    wh[q] = None
                gh[q] = start_gather(q, nxt)
            p = c % _NBUF
            gh[p].wait()
            wh[p] = start_write(p, c)
        for p in range(_NBUF):
            if wh[p] is not None:
                wh[p].wait()

    @pl.when(base + _ROWS_PER_W - 1 <= clamp_s)
    def _identity_slab():
        def start_linear(p, c):
            row0 = base + c * _CHUNK
            return pltpu.async_copy(table_hbm.at[pl.ds(row0, _CHUNK)],
                                    buf[p], gsem[p])

        run_slab(start_linear, prefetched=True)

    @pl.when(base + _ROWS_PER_W - 1 > clamp_s)
    def _clamped_slab():
        for h in hpre:
            h.wait()

        def start_indirect(p, c):
            row0 = base + c * _CHUNK
            for j in range(_CHUNK // _L):
                v = jnp.minimum(iota + (row0 + j * _L), clamp_vec)
                idx[p][pl.ds(j * _L, _L)] = jnp.maximum(v, 0)
            return pltpu.async_copy(table_hbm.at[idx[p]], buf[p], gsem[p])

        run_slab(start_indirect, prefetched=False)


@functools.partial(
    pl.kernel,
    out_type=jax.ShapeDtypeStruct((MAX_LEN, DIM), jnp.float32),
    mesh=plsc.VectorSubcoreMesh(core_axis_name="c", subcore_axis_name="s"),
    scratch_types=(
        [pltpu.VMEM((_L,), jnp.int32)]
        + [pltpu.VMEM((_CHUNK,), jnp.int32) for _ in range(_NBUF)]
        + [pltpu.VMEM((_CHUNK, DIM), jnp.float32) for _ in range(_NBUF)]
        + [pltpu.SemaphoreType.DMA for _ in range(2 * _NBUF)]
    ),
)
def _pe_call(clamp_hbm, table_hbm, out_hbm, clamp_v, *scratch):
    idx = scratch[:_NBUF]
    buf = scratch[_NBUF:2 * _NBUF]
    gsem = scratch[2 * _NBUF:3 * _NBUF]
    wsem = scratch[3 * _NBUF:4 * _NBUF]
    _pe_kernel(clamp_hbm, table_hbm, out_hbm, clamp_v, idx, buf, gsem, wsem)


def kernel(seq_len, table):
    clamp = jnp.full((_L,), jnp.asarray(seq_len, jnp.int32) - 1, jnp.int32)
    return _pe_call(clamp, table)


# speedup vs baseline: 1.2535x; 1.2086x over previous
import jax
import jax.numpy as jnp
from jax import lax
from jax.experimental import pallas as pl
from jax.experimental.pallas import tpu as pltpu

MAX_LEN = 8192
DIM = 1024
_BT = 256


def _tc_body(clamp_ref, last_ref, x_ref, o_ref):
    i = pl.program_id(0)
    c = clamp_ref[0]
    rows = i * _BT + lax.broadcasted_iota(jnp.int32, (_BT, 1), 0)
    o_ref[...] = jnp.where(rows <= c, x_ref[...], last_ref[...])


_tc_call = pl.pallas_call(
    _tc_body,
    grid=(MAX_LEN // _BT,),
    in_specs=[
        pl.BlockSpec(memory_space=pltpu.SMEM),
        pl.BlockSpec((1, DIM), lambda i: (0, 0)),
        pl.BlockSpec((_BT, DIM), lambda i: (i, 0)),
    ],
    out_specs=pl.BlockSpec((_BT, DIM), lambda i: (i, 0)),
    out_shape=jax.ShapeDtypeStruct((MAX_LEN, DIM), jnp.float32),
)


def kernel(seq_len, table):
    s = jnp.asarray(seq_len, jnp.int32)
    clamp = s - 1
    safe = jnp.clip(clamp, 0, MAX_LEN - 1)
    last_row = lax.dynamic_slice(table, (safe, jnp.int32(0)), (1, DIM))
    return _tc_call(clamp.reshape(1), last_row, table)
